# baseline (speedup 1.0000x reference)
import functools

import jax
import jax.numpy as jnp
from jax import lax
from jax.experimental import pallas as pl
from jax.experimental.pallas import tpu as pltpu
from jax.experimental.pallas import tpu_sc as plsc

NUM_EXPERTS = 32
D_MODEL = 2880
TOP_K = 4
BT = 4096
NW = 32
L = 16


def _keys_body(hs_ref, w_ref, b_ref, keys_ref):
    logits32 = jax.lax.dot_general(
        w_ref[...], hs_ref[...], (((1,), (0,)), ((), ())),
        preferred_element_type=jnp.float32,
    )
    s32v = logits32 + b_ref[...].astype(jnp.float32)
    v = jax.lax.bitcast_convert_type(s32v, jnp.int32)
    x = (v & jnp.int32(0x7FFFFFFF)) ^ jax.lax.shift_right_arithmetic(v, 31)
    iota = jax.lax.broadcasted_iota(jnp.int32, x.shape, 0)
    keys_ref[...] = (x | jnp.int32(0xFFFF)) ^ iota


def _make_sc_router(n_tokens):
    n_per_w = n_tokens // NW
    n_groups = n_per_w // L
    mesh = plsc.VectorSubcoreMesh(core_axis_name="c", subcore_axis_name="s")

    @functools.partial(
        pl.kernel, mesh=mesh,
        out_type=[
            jax.ShapeDtypeStruct((NUM_EXPERTS, n_tokens), jnp.float32),
            jax.ShapeDtypeStruct((TOP_K, n_tokens), jnp.int32),
        ],
        scratch_types=[
            pltpu.VMEM((NUM_EXPERTS, n_per_w), jnp.int32),
            pltpu.VMEM((NUM_EXPERTS, n_per_w), jnp.float32),
            pltpu.VMEM((TOP_K, n_per_w), jnp.int32),
        ],
    )
    def sc_router(keys_hbm, scores_hbm, idx_hbm, keys_v, scores_v, idx_v):
        wid = lax.axis_index("s") * 2 + lax.axis_index("c")
        base = wid * n_per_w
        pltpu.sync_copy(keys_hbm.at[:, pl.ds(base, n_per_w)], keys_v)

        int_min = jnp.full((L,), -2147483648, jnp.int32)
        lane = lax.broadcasted_iota(jnp.int32, (L,), 0)

        def group_body(g, carry):
            t0 = g * L
            m1, m2, m3, m4 = int_min, int_min, int_min, int_min
            for e in range(NUM_EXPERTS):
                v = keys_v[e, pl.ds(t0, L)]
                hi = jnp.maximum(m1, v)
                lo = jnp.minimum(m1, v)
                m1 = hi
                hi = jnp.maximum(m2, lo)
                lo = jnp.minimum(m2, lo)
                m2 = hi
                hi = jnp.maximum(m3, lo)
                lo = jnp.minimum(m3, lo)
                m3 = hi
                m4 = jnp.maximum(m4, lo)

            ms = (m1, m2, m3, m4)
            idxs = [(m ^ jnp.int32(0xFFFF)) & jnp.int32(0xFFFF) for m in ms]
            vals = []
            for m in ms:
                y = (m & jnp.int32(0x7FFFFFFF)) ^ lax.shift_right_arithmetic(m, 31)
                vals.append(lax.bitcast_convert_type(y & jnp.int32(-65536),
                                                     jnp.float32))
            es = [jnp.exp(val - vals[0]) for val in vals]
            ssum = (es[0] + es[1]) + (es[2] + es[3])
            ps = [e / ssum for e in es]

            zero = jnp.zeros((L,), jnp.float32)
            for j in range(TOP_K):
                idx_v[j, pl.ds(t0, L)] = idxs[j]
            for e in range(NUM_EXPERTS):
                s = zero
                for j in range(TOP_K):
                    s = jnp.where(idxs[j] == e, ps[j], s)
                scores_v[e, pl.ds(t0, L)] = s
            return carry

        lax.fori_loop(0, n_groups, group_body, 0)
        pltpu.sync_copy(scores_v, scores_hbm.at[:, pl.ds(base, n_per_w)])
        pltpu.sync_copy(idx_v, idx_hbm.at[:, pl.ds(base, n_per_w)])

    return sc_router


def kernel(hidden_states, weight, bias):
    hs_t = hidden_states.reshape(-1, D_MODEL).T
    n_tokens = hs_t.shape[1]
    grid = (n_tokens // BT,)
    bias2 = bias.reshape(NUM_EXPERTS, 1)
    keys = pl.pallas_call(
        _keys_body,
        grid=grid,
        in_specs=[
            pl.BlockSpec((D_MODEL, BT), lambda i: (0, i)),
            pl.BlockSpec((NUM_EXPERTS, D_MODEL), lambda i: (0, 0)),
            pl.BlockSpec((NUM_EXPERTS, 1), lambda i: (0, 0)),
        ],
        out_specs=pl.BlockSpec((NUM_EXPERTS, BT), lambda i: (0, i)),
        out_shape=jax.ShapeDtypeStruct((NUM_EXPERTS, n_tokens), jnp.int32),
        compiler_params=pltpu.CompilerParams(
            dimension_semantics=("arbitrary",),
        ),
    )(hs_t, weight, bias2)

    scores_t, idx_t = _make_sc_router(n_tokens)(keys)
    return scores_t.T.astype(jnp.bfloat16), idx_t.T
